# baseline (speedup 1.0000x reference)
import functools

import jax
import jax.numpy as jnp
import numpy as np
from jax import lax
from jax.experimental import pallas as pl
from jax.experimental.pallas import tpu as pltpu
from jax.experimental.pallas import tpu_sc as plsc

_FIELD_DIMS = np.full(27, 40000, dtype=np.int64)
_USED = _FIELD_DIMS[:-1]
_NF = len(_USED)
_D = 16
_VOCAB = int(_USED.sum())
_OFFSETS = np.concatenate([[0], np.cumsum(_USED)[:-1]]).astype(np.int32)
_B = 16384

_NW = 32
_BPW = _B // _NW
_C = 64
_NCHUNK = _BPW // _C
_CI = _C * _NF

_VB = _VOCAB // 128
_CB = 125


_NCB = (_VB + _NW - 1) // _NW
_CPS = 8
_NCHT = _NCB // _CPS
_ROWS = _CPS * 128
_ROWD = 128 * _D


def _transpose_blocks(vin, vout, idxs8, ncc):
    def ld(cc, w0, d):
        return vin[d // 8, cc, d % 8, pl.ds(w0 * 16, 16)]

    def sc(cc, w0, d, vec):
        off = cc * 2048 + w0 * 256 + 8 * (d // 8)
        plsc.store_scatter(vout.at[pl.ds(off, 256)], [idxs8[d % 8]], vec)

    groups = [(cc, w0) for cc in range(ncc) for w0 in range(8)]
    prev = [ld(*groups[0], d) for d in range(16)]
    pg = groups[0]
    for gi in range(1, len(groups)):
        cur = []
        for d in range(16):
            sc(pg[0], pg[1], d, prev[d])
            cur.append(ld(*groups[gi], d))
        prev, pg = cur, groups[gi]
    for d in range(16):
        sc(pg[0], pg[1], d, prev[d])


def _sc_transpose_body(v4_hbm, t4_hbm, out_hbm, tout_hbm,
                       vin0, vin1, vout0, vout1,
                       si0, si1, so0, so1):
    nc = 2
    wid = lax.axis_index("s") * nc + lax.axis_index("c")
    wc0 = wid * _NCB
    lanes16 = lax.iota(jnp.int32, 16) * 16
    idxs8 = [lanes16 + d for d in range(8)]
    vins = (vin0, vin1)
    vouts = (vout0, vout1)
    sis = (si0, si1)
    sos = (so0, so1)
    nch = _NCHT + 1

    def c0_of(j):
        return jnp.minimum(wc0 + j * _CPS, _VB - _CPS)

    pltpu.async_copy(v4_hbm.at[:, pl.ds(c0_of(0), _CPS), :, :], vin0, si0)
    pltpu.async_copy(v4_hbm.at[:, pl.ds(c0_of(1), _CPS), :, :], vin1, si1)

    def step(k, _):
        for par in range(2):
            j = 2 * k + par
            c0 = c0_of(j)
            vin = vins[par]
            vout = vouts[par]
            pltpu.make_async_copy(
                v4_hbm.at[:, pl.ds(c0, _CPS), :, :], vin, sis[par]).wait()

            @pl.when(k > 0)
            def _():
                pltpu.make_async_copy(
                    vout.at[pl.ds(0, _ROWD * _CPS)],
                    out_hbm.at[pl.ds(c0 * _ROWD, _ROWD * _CPS)],
                    sos[par]).wait()

            _transpose_blocks(vin, vout, idxs8, _CPS)

            pltpu.async_copy(
                vout.at[pl.ds(0, _ROWD * _CPS)],
                out_hbm.at[pl.ds(c0 * _ROWD, _ROWD * _CPS)], sos[par])

            @pl.when(j + 2 < nch)
            def _():
                pltpu.async_copy(
                    v4_hbm.at[:, pl.ds(c0_of(j + 2), _CPS), :, :],
                    vin, sis[par])

        return 0

    lax.fori_loop(0, nch // 2, step, 0)
    pltpu.make_async_copy(
        vout0.at[pl.ds(0, _ROWD * _CPS)],
        out_hbm.at[pl.ds(0, _ROWD * _CPS)], so0).wait()
    pltpu.make_async_copy(
        vout1.at[pl.ds(0, _ROWD * _CPS)],
        out_hbm.at[pl.ds(0, _ROWD * _CPS)], so1).wait()

    pltpu.sync_copy(t4_hbm.at[:, pl.ds(wid * 4, 4), :, :],
                    vin0.at[:, pl.ds(0, 4), :, :])
    _transpose_blocks(vin0, vout0, idxs8, 4)
    pltpu.sync_copy(vout0.at[pl.ds(0, 4 * _ROWD)],
                    tout_hbm.at[pl.ds(wid * 4 * _ROWD, 4 * _ROWD)])


def _sc_body(xT_hbm, t_hbm, emb_hbm, fc_hbm, lw_hbm, bias_hbm,
             out_hbm,
             xv0, xv1, ib0, ib1, rows0, rows1, fcv0, fcv1, tv0, tv1,
             oo0, oo1, wbuf, lw_v, bias_v,
             se0, se1, sf0, sf1, so0, so1):
    nc = 2
    wid = lax.axis_index("s") * nc + lax.axis_index("c")

    pltpu.sync_copy(lw_hbm, lw_v)
    pltpu.sync_copy(bias_hbm, bias_v)
    lwv = lw_v[0, :]
    bv = bias_v[0, :]
    lanes = lax.iota(jnp.int32, 16)
    lanes16 = lanes * 16
    idxs2 = [lanes16 + d for d in range(16)]
    xv = (xv0, xv1)
    ib = (ib0, ib1)
    rowsv = (rows0, rows1)
    fcv = (fcv0, fcv1)
    tv = (tv0, tv1)
    oov = (oo0, oo1)
    ses = (se0, se1)
    sfs = (sf0, sf1)
    sos = (so0, so1)

    def issue(i, par):
        b0 = wid * _BPW + i * _C
        pltpu.sync_copy(xT_hbm.at[:, pl.ds(b0, _C)], xv[par])

        def build(f, _):
            off = f * 40000
            base = pl.multiple_of(f * _C, 16)
            for g in range(_C // 16):
                ib[par][pl.ds(base + g * 16, 16)] = (
                    xv[par][f, pl.ds(g * 16, 16)] + off)
            return 0

        lax.fori_loop(0, _NF, build, 0)
        pltpu.sync_copy(t_hbm.at[pl.ds(b0, _C), :], tv[par])
        pltpu.async_copy(emb_hbm.at[ib[par]], rowsv[par], ses[par])
        pltpu.async_copy(fc_hbm.at[ib[par]], fcv[par], sfs[par])

    issue(0, 0)
    for i in range(_NCHUNK):
        par = i % 2
        b0 = wid * _BPW + i * _C
        rv = rowsv[par]
        fv = fcv[par]
        tb_v = tv[par]
        pltpu.make_async_copy(emb_hbm.at[ib[par]], rv, ses[par]).wait()
        pltpu.make_async_copy(fc_hbm.at[ib[par]], fv, sfs[par]).wait()
        if i + 1 < _NCHUNK:
            issue(i + 1, 1 - par)
        if i >= 2:
            pltpu.make_async_copy(
                oov[par], out_hbm.at[pl.ds(b0, _C)], sos[par]).wait()

        for g in range(_C // 16):
            def row_body(j, _):
                c = g * 16 + j
                tb = tb_v[c, :]
                s = tb
                q = tb * tb
                for f in range(_NF):
                    v = rv[f * _C + c, :]
                    s = s + v
                    q = q + v * v
                w = 0.5 * (s * s - q) + tb * lwv
                wbuf[pl.ds(j * 16, 16)] = w
                return 0

            lax.fori_loop(0, 16, row_body, 0)
            acc = plsc.load_gather(wbuf, [idxs2[0]])
            for d in range(1, 16):
                acc = acc + plsc.load_gather(wbuf, [idxs2[d]])
            for f in range(_NF):
                acc = acc + fv[pl.ds(f * _C + g * 16, 16)]
            oov[par][pl.ds(g * 16, 16)] = acc + bv

        pltpu.async_copy(oov[par], out_hbm.at[pl.ds(b0, _C)], sos[par])

    for par in range(2):
        pltpu.make_async_copy(
            oov[par], out_hbm.at[pl.ds(0, _C)], sos[par]).wait()


@functools.partial(jax.jit, static_argnames=())
def kernel(x, t, emb_table, fc_table, fc_bias, lin_w, lin_b):
    v4 = emb_table.reshape(_VB, 128, 2, 8).transpose(2, 0, 3, 1)
    t4 = t.reshape(128, 128, 2, 8).transpose(2, 0, 3, 1)
    mesh_t = plsc.VectorSubcoreMesh(core_axis_name="c", subcore_axis_name="s")
    emb_flat, t_flat = pl.kernel(
        _sc_transpose_body,
        out_type=(jax.ShapeDtypeStruct((_VOCAB * _D,), jnp.float32),
                  jax.ShapeDtypeStruct((_B * _D,), jnp.float32)),
        mesh=mesh_t,
        compiler_params=pltpu.CompilerParams(
            needs_layout_passes=False, use_tc_tiling_on_sc=False),
        scratch_types=[
            pltpu.VMEM((2, _CPS, 8, 128), jnp.float32),
            pltpu.VMEM((2, _CPS, 8, 128), jnp.float32),
            pltpu.VMEM((_ROWS * _D + 16,), jnp.float32),
            pltpu.VMEM((_ROWS * _D + 16,), jnp.float32),
            pltpu.SemaphoreType.DMA,
            pltpu.SemaphoreType.DMA,
            pltpu.SemaphoreType.DMA,
            pltpu.SemaphoreType.DMA,
        ],
    )(v4, t4)
    emb_rm = emb_flat.reshape(_VOCAB, _D)
    t_rm = t_flat.reshape(_B, _D)
    bias_vec = jnp.broadcast_to((fc_bias + lin_b).reshape(1, 1), (1, _D))

    mesh = plsc.VectorSubcoreMesh(core_axis_name="c", subcore_axis_name="s")
    fn = pl.kernel(
        _sc_body,
        out_type=jax.ShapeDtypeStruct((_B,), jnp.float32),
        mesh=mesh,
        compiler_params=pltpu.CompilerParams(
            needs_layout_passes=False, use_tc_tiling_on_sc=False),
        scratch_types=[
            pltpu.VMEM((_NF, _C), jnp.int32),
            pltpu.VMEM((_NF, _C), jnp.int32),
            pltpu.VMEM((_CI,), jnp.int32),
            pltpu.VMEM((_CI,), jnp.int32),
            pltpu.VMEM((_CI, _D), jnp.float32),
            pltpu.VMEM((_CI, _D), jnp.float32),
            pltpu.VMEM((_CI,), jnp.float32),
            pltpu.VMEM((_CI,), jnp.float32),
            pltpu.VMEM((_C, _D), jnp.float32),
            pltpu.VMEM((_C, _D), jnp.float32),
            pltpu.VMEM((_C,), jnp.float32),
            pltpu.VMEM((_C,), jnp.float32),
            pltpu.VMEM((256,), jnp.float32),
            pltpu.VMEM((1, _D), jnp.float32),
            pltpu.VMEM((1, _D), jnp.float32),
            pltpu.SemaphoreType.DMA,
            pltpu.SemaphoreType.DMA,
            pltpu.SemaphoreType.DMA,
            pltpu.SemaphoreType.DMA,
            pltpu.SemaphoreType.DMA,
            pltpu.SemaphoreType.DMA,
        ],
    )
    return fn(x.T, t_rm, emb_rm, jnp.squeeze(fc_table, 1), lin_w, bias_vec)


# speedup vs baseline: 1.0663x; 1.0663x over previous
import functools

import jax
import jax.numpy as jnp
import numpy as np
from jax import lax
from jax.experimental import pallas as pl
from jax.experimental.pallas import tpu as pltpu
from jax.experimental.pallas import tpu_sc as plsc

_FIELD_DIMS = np.full(27, 40000, dtype=np.int64)
_USED = _FIELD_DIMS[:-1]
_NF = len(_USED)
_D = 16
_VOCAB = int(_USED.sum())
_OFFSETS = np.concatenate([[0], np.cumsum(_USED)[:-1]]).astype(np.int32)
_B = 16384

_NW = 32
_BPW = _B // _NW
_C = 64
_NCHUNK = _BPW // _C
_CI = _C * _NF

_VB = _VOCAB // 128
_CB = 125


_NCB = (_VB + _NW - 1) // _NW
_CPS = 8
_NCHT = _NCB // _CPS
_ROWS = _CPS * 128
_ROWD = 128 * _D


def _transpose_blocks(vin, vout, idxs8, ncc):
    def ld(cc, w0, d):
        return vin[d // 8, cc, d % 8, pl.ds(w0 * 16, 16)]

    def sc(cc, w0, d, vec):
        off = cc * 2048 + w0 * 256 + 8 * (d // 8)
        plsc.store_scatter(vout.at[pl.ds(off, 256)], [idxs8[d % 8]], vec)

    groups = [(cc, w0) for cc in range(ncc) for w0 in range(8)]
    prev = [ld(*groups[0], d) for d in range(16)]
    pg = groups[0]
    for gi in range(1, len(groups)):
        cur = []
        for d in range(16):
            sc(pg[0], pg[1], d, prev[d])
            cur.append(ld(*groups[gi], d))
        prev, pg = cur, groups[gi]
    for d in range(16):
        sc(pg[0], pg[1], d, prev[d])


def _sc_transpose_body(v4_hbm, t4_hbm, out_hbm, tout_hbm,
                       vin0, vin1, vout0, vout1,
                       si0, si1, so0, so1):
    nc = 2
    wid = lax.axis_index("s") * nc + lax.axis_index("c")
    wc0 = wid * _NCB
    lanes16 = lax.iota(jnp.int32, 16) * 16
    idxs8 = [lanes16 + d for d in range(8)]
    vins = (vin0, vin1)
    vouts = (vout0, vout1)
    sis = (si0, si1)
    sos = (so0, so1)
    nch = _NCHT + 1

    def c0_of(j):
        return jnp.minimum(wc0 + j * _CPS, _VB - _CPS)

    pltpu.async_copy(v4_hbm.at[:, pl.ds(c0_of(0), _CPS), :, :], vin0, si0)
    pltpu.async_copy(v4_hbm.at[:, pl.ds(c0_of(1), _CPS), :, :], vin1, si1)

    def step(k, _):
        for par in range(2):
            j = 2 * k + par
            c0 = c0_of(j)
            vin = vins[par]
            vout = vouts[par]
            pltpu.make_async_copy(
                v4_hbm.at[:, pl.ds(c0, _CPS), :, :], vin, sis[par]).wait()

            @pl.when(k > 0)
            def _():
                pltpu.make_async_copy(
                    vout.at[pl.ds(0, _ROWD * _CPS)],
                    out_hbm.at[pl.ds(c0 * _ROWD, _ROWD * _CPS)],
                    sos[par]).wait()

            _transpose_blocks(vin, vout, idxs8, _CPS)

            pltpu.async_copy(
                vout.at[pl.ds(0, _ROWD * _CPS)],
                out_hbm.at[pl.ds(c0 * _ROWD, _ROWD * _CPS)], sos[par])

            @pl.when(j + 2 < nch)
            def _():
                pltpu.async_copy(
                    v4_hbm.at[:, pl.ds(c0_of(j + 2), _CPS), :, :],
                    vin, sis[par])

        return 0

    lax.fori_loop(0, nch // 2, step, 0)
    pltpu.make_async_copy(
        vout0.at[pl.ds(0, _ROWD * _CPS)],
        out_hbm.at[pl.ds(0, _ROWD * _CPS)], so0).wait()
    pltpu.make_async_copy(
        vout1.at[pl.ds(0, _ROWD * _CPS)],
        out_hbm.at[pl.ds(0, _ROWD * _CPS)], so1).wait()

    pltpu.sync_copy(t4_hbm.at[:, pl.ds(wid * 4, 4), :, :],
                    vin0.at[:, pl.ds(0, 4), :, :])
    _transpose_blocks(vin0, vout0, idxs8, 4)
    pltpu.sync_copy(vout0.at[pl.ds(0, 4 * _ROWD)],
                    tout_hbm.at[pl.ds(wid * 4 * _ROWD, 4 * _ROWD)])


def _sc_body(xT_hbm, t_hbm, emb_hbm, fc_hbm, lw_hbm, bias_hbm,
             out_hbm,
             xv0, xv1, ib0, ib1, rows0, rows1, fcv0, fcv1, tv0, tv1,
             oo0, oo1, wbuf, lw_v, bias_v,
             se0, se1, sf0, sf1, so0, so1, sx0, sx1, st0, st1):
    nc = 2
    wid = lax.axis_index("s") * nc + lax.axis_index("c")

    pltpu.sync_copy(lw_hbm, lw_v)
    pltpu.sync_copy(bias_hbm, bias_v)
    lwv = lw_v[0, :]
    bv = bias_v[0, :]
    lanes = lax.iota(jnp.int32, 16)
    lanes16 = lanes * 16
    idxs2 = [lanes16 + d for d in range(16)]
    xv = (xv0, xv1)
    ib = (ib0, ib1)
    rowsv = (rows0, rows1)
    fcv = (fcv0, fcv1)
    tv = (tv0, tv1)
    oov = (oo0, oo1)
    ses = (se0, se1)
    sfs = (sf0, sf1)
    sos = (so0, so1)
    sxs = (sx0, sx1)
    sts = (st0, st1)

    def stage_x(i, par):
        b0 = wid * _BPW + i * _C
        pltpu.async_copy(xT_hbm.at[:, pl.ds(b0, _C)], xv[par], sxs[par])

    def stage_t(i, par):
        b0 = wid * _BPW + i * _C
        pltpu.async_copy(t_hbm.at[pl.ds(b0, _C), :], tv[par], sts[par])

    def fire(i, par):
        b0 = wid * _BPW + i * _C
        pltpu.make_async_copy(
            xT_hbm.at[:, pl.ds(b0, _C)], xv[par], sxs[par]).wait()

        def build(f, _):
            off = f * 40000
            base = pl.multiple_of(f * _C, 16)
            vs = [xv[par][f, pl.ds(g * 16, 16)] + off
                  for g in range(_C // 16)]
            for g in range(_C // 16):
                ib[par][pl.ds(base + g * 16, 16)] = vs[g]
            return 0

        lax.fori_loop(0, _NF, build, 0)
        pltpu.async_copy(emb_hbm.at[ib[par]], rowsv[par], ses[par])
        pltpu.async_copy(fc_hbm.at[ib[par]], fcv[par], sfs[par])

    stage_x(0, 0)
    stage_t(0, 0)
    fire(0, 0)
    stage_x(1, 1)
    stage_t(1, 1)
    for i in range(_NCHUNK):
        par = i % 2
        b0 = wid * _BPW + i * _C
        rv = rowsv[par]
        fv = fcv[par]
        tb_v = tv[par]
        pltpu.make_async_copy(emb_hbm.at[ib[par]], rv, ses[par]).wait()
        pltpu.make_async_copy(fc_hbm.at[ib[par]], fv, sfs[par]).wait()
        pltpu.make_async_copy(
            t_hbm.at[pl.ds(b0, _C), :], tb_v, sts[par]).wait()
        if i + 1 < _NCHUNK:
            fire(i + 1, 1 - par)
        if i + 2 < _NCHUNK:
            stage_x(i + 2, par)
        if i >= 2:
            pltpu.make_async_copy(
                oov[par], out_hbm.at[pl.ds(b0, _C)], sos[par]).wait()

        for g in range(_C // 16):
            def row_body(j, _):
                c = g * 16 + j
                tb = tb_v[c, :]
                s = tb
                q = tb * tb
                for f in range(_NF):
                    v = rv[f * _C + c, :]
                    s = s + v
                    q = q + v * v
                w = 0.5 * (s * s - q) + tb * lwv
                wbuf[pl.ds(j * 16, 16)] = w
                return 0

            lax.fori_loop(0, 16, row_body, 0)
            acc = plsc.load_gather(wbuf, [idxs2[0]])
            for d in range(1, 16):
                acc = acc + plsc.load_gather(wbuf, [idxs2[d]])
            for f in range(_NF):
                acc = acc + fv[pl.ds(f * _C + g * 16, 16)]
            oov[par][pl.ds(g * 16, 16)] = acc + bv

        if i + 2 < _NCHUNK:
            stage_t(i + 2, par)
        pltpu.async_copy(oov[par], out_hbm.at[pl.ds(b0, _C)], sos[par])

    for par in range(2):
        pltpu.make_async_copy(
            oov[par], out_hbm.at[pl.ds(0, _C)], sos[par]).wait()


@functools.partial(jax.jit, static_argnames=())
def kernel(x, t, emb_table, fc_table, fc_bias, lin_w, lin_b):
    v4 = emb_table.reshape(_VB, 128, 2, 8).transpose(2, 0, 3, 1)
    t4 = t.reshape(128, 128, 2, 8).transpose(2, 0, 3, 1)
    mesh_t = plsc.VectorSubcoreMesh(core_axis_name="c", subcore_axis_name="s")
    emb_flat, t_flat = pl.kernel(
        _sc_transpose_body,
        out_type=(jax.ShapeDtypeStruct((_VOCAB * _D,), jnp.float32),
                  jax.ShapeDtypeStruct((_B * _D,), jnp.float32)),
        mesh=mesh_t,
        compiler_params=pltpu.CompilerParams(
            needs_layout_passes=False, use_tc_tiling_on_sc=False),
        scratch_types=[
            pltpu.VMEM((2, _CPS, 8, 128), jnp.float32),
            pltpu.VMEM((2, _CPS, 8, 128), jnp.float32),
            pltpu.VMEM((_ROWS * _D + 16,), jnp.float32),
            pltpu.VMEM((_ROWS * _D + 16,), jnp.float32),
            pltpu.SemaphoreType.DMA,
            pltpu.SemaphoreType.DMA,
            pltpu.SemaphoreType.DMA,
            pltpu.SemaphoreType.DMA,
        ],
    )(v4, t4)
    emb_rm = emb_flat.reshape(_VOCAB, _D)
    t_rm = t_flat.reshape(_B, _D)
    bias_vec = jnp.broadcast_to((fc_bias + lin_b).reshape(1, 1), (1, _D))

    mesh = plsc.VectorSubcoreMesh(core_axis_name="c", subcore_axis_name="s")
    fn = pl.kernel(
        _sc_body,
        out_type=jax.ShapeDtypeStruct((_B,), jnp.float32),
        mesh=mesh,
        compiler_params=pltpu.CompilerParams(
            needs_layout_passes=False, use_tc_tiling_on_sc=False),
        scratch_types=[
            pltpu.VMEM((_NF, _C), jnp.int32),
            pltpu.VMEM((_NF, _C), jnp.int32),
            pltpu.VMEM((_CI,), jnp.int32),
            pltpu.VMEM((_CI,), jnp.int32),
            pltpu.VMEM((_CI, _D), jnp.float32),
            pltpu.VMEM((_CI, _D), jnp.float32),
            pltpu.VMEM((_CI,), jnp.float32),
            pltpu.VMEM((_CI,), jnp.float32),
            pltpu.VMEM((_C, _D), jnp.float32),
            pltpu.VMEM((_C, _D), jnp.float32),
            pltpu.VMEM((_C,), jnp.float32),
            pltpu.VMEM((_C,), jnp.float32),
            pltpu.VMEM((256,), jnp.float32),
            pltpu.VMEM((1, _D), jnp.float32),
            pltpu.VMEM((1, _D), jnp.float32),
            pltpu.SemaphoreType.DMA,
            pltpu.SemaphoreType.DMA,
            pltpu.SemaphoreType.DMA,
            pltpu.SemaphoreType.DMA,
            pltpu.SemaphoreType.DMA,
            pltpu.SemaphoreType.DMA,
            pltpu.SemaphoreType.DMA,
            pltpu.SemaphoreType.DMA,
            pltpu.SemaphoreType.DMA,
            pltpu.SemaphoreType.DMA,
        ],
    )
    return fn(x.T, t_rm, emb_rm, jnp.squeeze(fc_table, 1), lin_w, bias_vec)


# speedup vs baseline: 1.0678x; 1.0014x over previous
import functools

import jax
import jax.numpy as jnp
import numpy as np
from jax import lax
from jax.experimental import pallas as pl
from jax.experimental.pallas import tpu as pltpu
from jax.experimental.pallas import tpu_sc as plsc

_FIELD_DIMS = np.full(27, 40000, dtype=np.int64)
_USED = _FIELD_DIMS[:-1]
_NF = len(_USED)
_D = 16
_VOCAB = int(_USED.sum())
_OFFSETS = np.concatenate([[0], np.cumsum(_USED)[:-1]]).astype(np.int32)
_B = 16384

_NW = 32
_BPW = _B // _NW
_C = 64
_NCHUNK = _BPW // _C
_CI = _C * _NF

_VB = _VOCAB // 128
_CB = 125


_NCB = (_VB + _NW - 1) // _NW
_CPS = 8
_NCHT = _NCB // _CPS
_ROWS = _CPS * 128
_ROWD = 128 * _D


def _transpose_blocks(vin, vout, idxs8, ncc):
    def ld(cc, w0, d):
        return vin[d // 8, cc, d % 8, pl.ds(w0 * 16, 16)]

    def sc(cc, w0, d, vec):
        off = cc * 2048 + w0 * 256 + 8 * (d // 8)
        plsc.store_scatter(vout.at[pl.ds(off, 256)], [idxs8[d % 8]], vec)

    groups = [(cc, w0) for cc in range(ncc) for w0 in range(8)]
    prev = [ld(*groups[0], d) for d in range(16)]
    pg = groups[0]
    for gi in range(1, len(groups)):
        cur = []
        for d in range(16):
            sc(pg[0], pg[1], d, prev[d])
            cur.append(ld(*groups[gi], d))
        prev, pg = cur, groups[gi]
    for d in range(16):
        sc(pg[0], pg[1], d, prev[d])


def _sc_transpose_body(v4_hbm, t4_hbm, out_hbm, tout_hbm,
                       vin0, vin1, vin2, vin3, vout0, vout1,
                       si0, si1, si2, si3, so0, so1):
    nc = 2
    wid = lax.axis_index("s") * nc + lax.axis_index("c")
    wc0 = wid * _NCB
    lanes16 = lax.iota(jnp.int32, 16) * 16
    idxs8 = [lanes16 + d for d in range(8)]
    vins = (vin0, vin1, vin2, vin3)
    vouts = (vout0, vout1)
    sis = (si0, si1, si2, si3)
    sos = (so0, so1)
    nch = _NCHT + 1

    def c0_of(j):
        return jnp.minimum(wc0 + j * _CPS, _VB - _CPS)

    for p in range(4):
        pltpu.async_copy(v4_hbm.at[:, pl.ds(c0_of(p), _CPS), :, :],
                         vins[p], sis[p])

    def step(k, _):
        for par in range(4):
            j = 4 * k + par
            c0 = c0_of(j)
            vin = vins[par]
            vout = vouts[par % 2]
            so = sos[par % 2]
            pltpu.make_async_copy(
                v4_hbm.at[:, pl.ds(c0, _CPS), :, :], vin, sis[par]).wait()

            def drain():
                pltpu.make_async_copy(
                    vout.at[pl.ds(0, _ROWD * _CPS)],
                    out_hbm.at[pl.ds(c0 * _ROWD, _ROWD * _CPS)], so).wait()

            if par >= 2:
                drain()
            else:
                @pl.when(k > 0)
                def _():
                    drain()

            _transpose_blocks(vin, vout, idxs8, _CPS)

            pltpu.async_copy(
                vout.at[pl.ds(0, _ROWD * _CPS)],
                out_hbm.at[pl.ds(c0 * _ROWD, _ROWD * _CPS)], so)

            @pl.when(j + 4 < nch)
            def _():
                pltpu.async_copy(
                    v4_hbm.at[:, pl.ds(c0_of(j + 4), _CPS), :, :],
                    vin, sis[par])

        return 0

    lax.fori_loop(0, nch // 4, step, 0)
    pltpu.make_async_copy(
        vout0.at[pl.ds(0, _ROWD * _CPS)],
        out_hbm.at[pl.ds(0, _ROWD * _CPS)], so0).wait()
    pltpu.make_async_copy(
        vout1.at[pl.ds(0, _ROWD * _CPS)],
        out_hbm.at[pl.ds(0, _ROWD * _CPS)], so1).wait()

    pltpu.sync_copy(t4_hbm.at[:, pl.ds(wid * 4, 4), :, :],
                    vin0.at[:, pl.ds(0, 4), :, :])
    _transpose_blocks(vin0, vout0, idxs8, 4)
    pltpu.sync_copy(vout0.at[pl.ds(0, 4 * _ROWD)],
                    tout_hbm.at[pl.ds(wid * 4 * _ROWD, 4 * _ROWD)])


def _sc_body(xT_hbm, t_hbm, emb_hbm, fc_hbm, lw_hbm, bias_hbm,
             out_hbm,
             xv0, xv1, ib0, ib1, rows0, rows1, fcv0, fcv1, tv0, tv1,
             oo0, oo1, wbuf, lw_v, bias_v,
             se0, se1, sf0, sf1, so0, so1, sx0, sx1, st0, st1):
    nc = 2
    wid = lax.axis_index("s") * nc + lax.axis_index("c")

    pltpu.sync_copy(lw_hbm, lw_v)
    pltpu.sync_copy(bias_hbm, bias_v)
    lwv = lw_v[0, :]
    bv = bias_v[0, :]
    lanes = lax.iota(jnp.int32, 16)
    lanes16 = lanes * 16
    idxs2 = [lanes16 + d for d in range(16)]
    xv = (xv0, xv1)
    ib = (ib0, ib1)
    rowsv = (rows0, rows1)
    fcv = (fcv0, fcv1)
    tv = (tv0, tv1)
    oov = (oo0, oo1)
    ses = (se0, se1)
    sfs = (sf0, sf1)
    sos = (so0, so1)
    sxs = (sx0, sx1)
    sts = (st0, st1)

    def stage_x(i, par):
        b0 = wid * _BPW + i * _C
        pltpu.async_copy(xT_hbm.at[:, pl.ds(b0, _C)], xv[par], sxs[par])

    def stage_t(i, par):
        b0 = wid * _BPW + i * _C
        pltpu.async_copy(t_hbm.at[pl.ds(b0, _C), :], tv[par], sts[par])

    def fire(i, par):
        b0 = wid * _BPW + i * _C
        pltpu.make_async_copy(
            xT_hbm.at[:, pl.ds(b0, _C)], xv[par], sxs[par]).wait()

        def build(f, _):
            off = f * 40000
            base = pl.multiple_of(f * _C, 16)
            vs = [xv[par][f, pl.ds(g * 16, 16)] + off
                  for g in range(_C // 16)]
            for g in range(_C // 16):
                ib[par][pl.ds(base + g * 16, 16)] = vs[g]
            return 0

        lax.fori_loop(0, _NF, build, 0)
        pltpu.async_copy(emb_hbm.at[ib[par]], rowsv[par], ses[par])
        pltpu.async_copy(fc_hbm.at[ib[par]], fcv[par], sfs[par])

    stage_x(0, 0)
    stage_t(0, 0)
    fire(0, 0)
    stage_x(1, 1)
    stage_t(1, 1)
    for i in range(_NCHUNK):
        par = i % 2
        b0 = wid * _BPW + i * _C
        rv = rowsv[par]
        fv = fcv[par]
        tb_v = tv[par]
        pltpu.make_async_copy(emb_hbm.at[ib[par]], rv, ses[par]).wait()
        pltpu.make_async_copy(fc_hbm.at[ib[par]], fv, sfs[par]).wait()
        pltpu.make_async_copy(
            t_hbm.at[pl.ds(b0, _C), :], tb_v, sts[par]).wait()
        if i + 1 < _NCHUNK:
            fire(i + 1, 1 - par)
        if i + 2 < _NCHUNK:
            stage_x(i + 2, par)
        if i >= 2:
            pltpu.make_async_copy(
                oov[par], out_hbm.at[pl.ds(b0, _C)], sos[par]).wait()

        for g in range(_C // 16):
            def row_body(j, _):
                c = g * 16 + j
                tb = tb_v[c, :]
                s = tb
                q = tb * tb
                for f in range(_NF):
                    v = rv[f * _C + c, :]
                    s = s + v
                    q = q + v * v
                w = 0.5 * (s * s - q) + tb * lwv
                wbuf[pl.ds(j * 16, 16)] = w
                return 0

            lax.fori_loop(0, 16, row_body, 0)
            acc = plsc.load_gather(wbuf, [idxs2[0]])
            for d in range(1, 16):
                acc = acc + plsc.load_gather(wbuf, [idxs2[d]])
            for f in range(_NF):
                acc = acc + fv[pl.ds(f * _C + g * 16, 16)]
            oov[par][pl.ds(g * 16, 16)] = acc + bv

        if i + 2 < _NCHUNK:
            stage_t(i + 2, par)
        pltpu.async_copy(oov[par], out_hbm.at[pl.ds(b0, _C)], sos[par])

    for par in range(2):
        pltpu.make_async_copy(
            oov[par], out_hbm.at[pl.ds(0, _C)], sos[par]).wait()


@functools.partial(jax.jit, static_argnames=())
def kernel(x, t, emb_table, fc_table, fc_bias, lin_w, lin_b):
    v4 = emb_table.reshape(_VB, 128, 2, 8).transpose(2, 0, 3, 1)
    t4 = t.reshape(128, 128, 2, 8).transpose(2, 0, 3, 1)
    mesh_t = plsc.VectorSubcoreMesh(core_axis_name="c", subcore_axis_name="s")
    emb_flat, t_flat = pl.kernel(
        _sc_transpose_body,
        out_type=(jax.ShapeDtypeStruct((_VOCAB * _D,), jnp.float32),
                  jax.ShapeDtypeStruct((_B * _D,), jnp.float32)),
        mesh=mesh_t,
        compiler_params=pltpu.CompilerParams(
            needs_layout_passes=False, use_tc_tiling_on_sc=False),
        scratch_types=[
            pltpu.VMEM((2, _CPS, 8, 128), jnp.float32),
            pltpu.VMEM((2, _CPS, 8, 128), jnp.float32),
            pltpu.VMEM((2, _CPS, 8, 128), jnp.float32),
            pltpu.VMEM((2, _CPS, 8, 128), jnp.float32),
            pltpu.VMEM((_ROWS * _D + 16,), jnp.float32),
            pltpu.VMEM((_ROWS * _D + 16,), jnp.float32),
            pltpu.SemaphoreType.DMA,
            pltpu.SemaphoreType.DMA,
            pltpu.SemaphoreType.DMA,
            pltpu.SemaphoreType.DMA,
            pltpu.SemaphoreType.DMA,
            pltpu.SemaphoreType.DMA,
        ],
    )(v4, t4)
    emb_rm = emb_flat.reshape(_VOCAB, _D)
    t_rm = t_flat.reshape(_B, _D)
    bias_vec = jnp.broadcast_to((fc_bias + lin_b).reshape(1, 1), (1, _D))

    mesh = plsc.VectorSubcoreMesh(core_axis_name="c", subcore_axis_name="s")
    fn = pl.kernel(
        _sc_body,
        out_type=jax.ShapeDtypeStruct((_B,), jnp.float32),
        mesh=mesh,
        compiler_params=pltpu.CompilerParams(
            needs_layout_passes=False, use_tc_tiling_on_sc=False),
        scratch_types=[
            pltpu.VMEM((_NF, _C), jnp.int32),
            pltpu.VMEM((_NF, _C), jnp.int32),
            pltpu.VMEM((_CI,), jnp.int32),
            pltpu.VMEM((_CI,), jnp.int32),
            pltpu.VMEM((_CI, _D), jnp.float32),
            pltpu.VMEM((_CI, _D), jnp.float32),
            pltpu.VMEM((_CI,), jnp.float32),
            pltpu.VMEM((_CI,), jnp.float32),
            pltpu.VMEM((_C, _D), jnp.float32),
            pltpu.VMEM((_C, _D), jnp.float32),
            pltpu.VMEM((_C,), jnp.float32),
            pltpu.VMEM((_C,), jnp.float32),
            pltpu.VMEM((256,), jnp.float32),
            pltpu.VMEM((1, _D), jnp.float32),
            pltpu.VMEM((1, _D), jnp.float32),
            pltpu.SemaphoreType.DMA,
            pltpu.SemaphoreType.DMA,
            pltpu.SemaphoreType.DMA,
            pltpu.SemaphoreType.DMA,
            pltpu.SemaphoreType.DMA,
            pltpu.SemaphoreType.DMA,
            pltpu.SemaphoreType.DMA,
            pltpu.SemaphoreType.DMA,
            pltpu.SemaphoreType.DMA,
            pltpu.SemaphoreType.DMA,
        ],
    )
    return fn(x.T, t_rm, emb_rm, jnp.squeeze(fc_table, 1), lin_w, bias_vec)


# speedup vs baseline: 1.0938x; 1.0243x over previous
import functools

import jax
import jax.numpy as jnp
import numpy as np
from jax import lax
from jax.experimental import pallas as pl
from jax.experimental.pallas import tpu as pltpu
from jax.experimental.pallas import tpu_sc as plsc

_FIELD_DIMS = np.full(27, 40000, dtype=np.int64)
_USED = _FIELD_DIMS[:-1]
_NF = len(_USED)
_D = 16
_VOCAB = int(_USED.sum())
_OFFSETS = np.concatenate([[0], np.cumsum(_USED)[:-1]]).astype(np.int32)
_B = 16384

_NW = 32
_BPW = _B // _NW
_C = 64
_NCHUNK = _BPW // _C
_CI = _C * _NF

_VB = _VOCAB // 128
_CB = 125


_NCB = (_VB + _NW - 1) // _NW
_CPS = 8
_NCHT = _NCB // _CPS
_ROWS = _CPS * 128
_ROWD = 128 * _D


def _transpose_blocks(vin, vout, idxs8, ncc):
    def ld(cc, w0, d):
        return vin[d // 8, cc, d % 8, pl.ds(w0 * 16, 16)]

    def sc(cc, w0, d, vec):
        off = cc * 2048 + w0 * 256 + 8 * (d // 8)
        plsc.store_scatter(vout.at[pl.ds(off, 256)], [idxs8[d % 8]], vec)

    groups = [(cc, w0) for cc in range(ncc) for w0 in range(8)]
    prev = [ld(*groups[0], d) for d in range(16)]
    pg = groups[0]
    for gi in range(1, len(groups)):
        cur = []
        for d in range(16):
            sc(pg[0], pg[1], d, prev[d])
            cur.append(ld(*groups[gi], d))
        prev, pg = cur, groups[gi]
    for d in range(16):
        sc(pg[0], pg[1], d, prev[d])


def _sc_transpose_body(v4_hbm, t4_hbm, out_hbm, tout_hbm,
                       vin0, vin1, vin2, vin3, vout0, vout1,
                       si0, si1, si2, si3, so0, so1):
    nc = 2
    wid = lax.axis_index("s") * nc + lax.axis_index("c")
    wc0 = wid * _NCB
    lanes16 = lax.iota(jnp.int32, 16) * 16
    idxs8 = [lanes16 + d for d in range(8)]
    vins = (vin0, vin1, vin2, vin3)
    vouts = (vout0, vout1)
    sis = (si0, si1, si2, si3)
    sos = (so0, so1)
    nch = _NCHT + 1

    def c0_of(j):
        return jnp.minimum(wc0 + j * _CPS, _VB - _CPS)

    for p in range(4):
        pltpu.async_copy(v4_hbm.at[:, pl.ds(c0_of(p), _CPS), :, :],
                         vins[p], sis[p])

    def step(k, _):
        for par in range(4):
            j = 4 * k + par
            c0 = c0_of(j)
            vin = vins[par]
            vout = vouts[par % 2]
            so = sos[par % 2]
            pltpu.make_async_copy(
                v4_hbm.at[:, pl.ds(c0, _CPS), :, :], vin, sis[par]).wait()

            def drain():
                pltpu.make_async_copy(
                    vout.at[pl.ds(0, _ROWD * _CPS)],
                    out_hbm.at[pl.ds(c0 * _ROWD, _ROWD * _CPS)], so).wait()

            if par >= 2:
                drain()
            else:
                @pl.when(k > 0)
                def _():
                    drain()

            _transpose_blocks(vin, vout, idxs8, _CPS)

            pltpu.async_copy(
                vout.at[pl.ds(0, _ROWD * _CPS)],
                out_hbm.at[pl.ds(c0 * _ROWD, _ROWD * _CPS)], so)

            @pl.when(j + 4 < nch)
            def _():
                pltpu.async_copy(
                    v4_hbm.at[:, pl.ds(c0_of(j + 4), _CPS), :, :],
                    vin, sis[par])

        return 0

    lax.fori_loop(0, nch // 4, step, 0)
    pltpu.make_async_copy(
        vout0.at[pl.ds(0, _ROWD * _CPS)],
        out_hbm.at[pl.ds(0, _ROWD * _CPS)], so0).wait()
    pltpu.make_async_copy(
        vout1.at[pl.ds(0, _ROWD * _CPS)],
        out_hbm.at[pl.ds(0, _ROWD * _CPS)], so1).wait()

    pltpu.sync_copy(t4_hbm.at[:, pl.ds(wid * 4, 4), :, :],
                    vin0.at[:, pl.ds(0, 4), :, :])
    _transpose_blocks(vin0, vout0, idxs8, 4)
    pltpu.sync_copy(vout0.at[pl.ds(0, 4 * _ROWD)],
                    tout_hbm.at[pl.ds(wid * 4 * _ROWD, 4 * _ROWD)])


def _sc_body(xT_hbm, t_hbm, emb_hbm, fc_hbm, lw_hbm, bias_hbm,
             out_hbm,
             xv0, xv1, ib0, ib1, rows0, rows1, fcv0, fcv1, tv0, tv1,
             oo0, oo1, wbuf, lw_v, bias_v,
             se0, se1, sf0, sf1, so0, so1, sx0, sx1, st0, st1):
    nc = 2
    wid = lax.axis_index("s") * nc + lax.axis_index("c")

    pltpu.sync_copy(lw_hbm, lw_v)
    pltpu.sync_copy(bias_hbm, bias_v)
    lwv = lw_v[0, :]
    bv = bias_v[0, :]
    lanes = lax.iota(jnp.int32, 16)
    lanes16 = lanes * 16
    idxs2 = [lanes16 + d for d in range(16)]
    xv = (xv0, xv1)
    ib = (ib0, ib1)
    rowsv = (rows0, rows1)
    fcv = (fcv0, fcv1)
    tv = (tv0, tv1)
    oov = (oo0, oo1)
    ses = (se0, se1)
    sfs = (sf0, sf1)
    sos = (so0, so1)
    sxs = (sx0, sx1)
    sts = (st0, st1)

    def stage_x(i, par):
        b0 = wid * _BPW + i * _C
        pltpu.async_copy(xT_hbm.at[:, pl.ds(b0, _C)], xv[par], sxs[par])

    def stage_t(i, par):
        b0 = wid * _BPW + i * _C
        pltpu.async_copy(t_hbm.at[pl.ds(b0, _C), :], tv[par], sts[par])

    def fire(i, par):
        b0 = wid * _BPW + i * _C
        pltpu.make_async_copy(
            xT_hbm.at[:, pl.ds(b0, _C)], xv[par], sxs[par]).wait()

        def build(f, _):
            off = f * 40000
            base = pl.multiple_of(f * _C, 16)
            vs = [xv[par][f, pl.ds(g * 16, 16)] + off
                  for g in range(_C // 16)]
            for g in range(_C // 16):
                ib[par][pl.ds(base + g * 16, 16)] = vs[g]
            return 0

        lax.fori_loop(0, _NF, build, 0)
        pltpu.async_copy(emb_hbm.at[ib[par]], rowsv[par], ses[par])
        pltpu.async_copy(fc_hbm.at[ib[par]], fcv[par], sfs[par])

    stage_x(0, 0)
    stage_t(0, 0)
    fire(0, 0)
    stage_x(1, 1)
    stage_t(1, 1)

    def chunk(ko, _):
        for par in range(2):
            i = 2 * ko + par
            b0 = pl.multiple_of(wid * _BPW + i * _C, _C)
            rv = rowsv[par]
            fv = fcv[par]
            tb_v = tv[par]
            pltpu.make_async_copy(emb_hbm.at[ib[par]], rv, ses[par]).wait()
            pltpu.make_async_copy(fc_hbm.at[ib[par]], fv, sfs[par]).wait()
            pltpu.make_async_copy(
                t_hbm.at[pl.ds(b0, _C), :], tb_v, sts[par]).wait()

            @pl.when(i < _NCHUNK - 1)
            def _():
                fire(i + 1, 1 - par)

            @pl.when(i < _NCHUNK - 2)
            def _():
                stage_x(i + 2, par)

            @pl.when(i >= 2)
            def _():
                pltpu.make_async_copy(
                    oov[par], out_hbm.at[pl.ds(b0, _C)], sos[par]).wait()

            for g in range(_C // 16):
                def row_body(jj, _):
                    c = g * 16 + jj * 2
                    tb0 = tb_v[c, :]
                    tb1 = tb_v[c + 1, :]
                    s0, s1 = tb0, tb1
                    q0, q1 = tb0 * tb0, tb1 * tb1
                    for f in range(_NF):
                        v0 = rv[f * _C + c, :]
                        v1 = rv[f * _C + c + 1, :]
                        s0 = s0 + v0
                        q0 = q0 + v0 * v0
                        s1 = s1 + v1
                        q1 = q1 + v1 * v1
                    w0 = 0.5 * (s0 * s0 - q0) + tb0 * lwv
                    w1 = 0.5 * (s1 * s1 - q1) + tb1 * lwv
                    j16 = pl.multiple_of(jj * 32, 16)
                    wbuf[pl.ds(j16, 16)] = w0
                    wbuf[pl.ds(j16 + 16, 16)] = w1
                    return 0

                lax.fori_loop(0, 8, row_body, 0)
                acc = plsc.load_gather(wbuf, [idxs2[0]])
                for d in range(1, 16):
                    acc = acc + plsc.load_gather(wbuf, [idxs2[d]])
                for f in range(_NF):
                    acc = acc + fv[pl.ds(f * _C + g * 16, 16)]
                oov[par][pl.ds(g * 16, 16)] = acc + bv

            @pl.when(i < _NCHUNK - 2)
            def _():
                stage_t(i + 2, par)

            pltpu.async_copy(oov[par], out_hbm.at[pl.ds(b0, _C)], sos[par])

        return 0

    lax.fori_loop(0, _NCHUNK // 2, chunk, 0)
    for par in range(2):
        pltpu.make_async_copy(
            oov[par], out_hbm.at[pl.ds(0, _C)], sos[par]).wait()


@functools.partial(jax.jit, static_argnames=())
def kernel(x, t, emb_table, fc_table, fc_bias, lin_w, lin_b):
    v4 = emb_table.reshape(_VB, 128, 2, 8).transpose(2, 0, 3, 1)
    t4 = t.reshape(128, 128, 2, 8).transpose(2, 0, 3, 1)
    mesh_t = plsc.VectorSubcoreMesh(core_axis_name="c", subcore_axis_name="s")
    emb_flat, t_flat = pl.kernel(
        _sc_transpose_body,
        out_type=(jax.ShapeDtypeStruct((_VOCAB * _D,), jnp.float32),
                  jax.ShapeDtypeStruct((_B * _D,), jnp.float32)),
        mesh=mesh_t,
        compiler_params=pltpu.CompilerParams(
            needs_layout_passes=False, use_tc_tiling_on_sc=False),
        scratch_types=[
            pltpu.VMEM((2, _CPS, 8, 128), jnp.float32),
            pltpu.VMEM((2, _CPS, 8, 128), jnp.float32),
            pltpu.VMEM((2, _CPS, 8, 128), jnp.float32),
            pltpu.VMEM((2, _CPS, 8, 128), jnp.float32),
            pltpu.VMEM((_ROWS * _D + 16,), jnp.float32),
            pltpu.VMEM((_ROWS * _D + 16,), jnp.float32),
            pltpu.SemaphoreType.DMA,
            pltpu.SemaphoreType.DMA,
            pltpu.SemaphoreType.DMA,
            pltpu.SemaphoreType.DMA,
            pltpu.SemaphoreType.DMA,
            pltpu.SemaphoreType.DMA,
        ],
    )(v4, t4)
    emb_rm = emb_flat.reshape(_VOCAB, _D)
    t_rm = t_flat.reshape(_B, _D)
    bias_vec = jnp.broadcast_to((fc_bias + lin_b).reshape(1, 1), (1, _D))

    mesh = plsc.VectorSubcoreMesh(core_axis_name="c", subcore_axis_name="s")
    fn = pl.kernel(
        _sc_body,
        out_type=jax.ShapeDtypeStruct((_B,), jnp.float32),
        mesh=mesh,
        compiler_params=pltpu.CompilerParams(
            needs_layout_passes=False, use_tc_tiling_on_sc=False),
        scratch_types=[
            pltpu.VMEM((_NF, _C), jnp.int32),
            pltpu.VMEM((_NF, _C), jnp.int32),
            pltpu.VMEM((_CI,), jnp.int32),
            pltpu.VMEM((_CI,), jnp.int32),
            pltpu.VMEM((_CI, _D), jnp.float32),
            pltpu.VMEM((_CI, _D), jnp.float32),
            pltpu.VMEM((_CI,), jnp.float32),
            pltpu.VMEM((_CI,), jnp.float32),
            pltpu.VMEM((_C, _D), jnp.float32),
            pltpu.VMEM((_C, _D), jnp.float32),
            pltpu.VMEM((_C,), jnp.float32),
            pltpu.VMEM((_C,), jnp.float32),
            pltpu.VMEM((256,), jnp.float32),
            pltpu.VMEM((1, _D), jnp.float32),
            pltpu.VMEM((1, _D), jnp.float32),
            pltpu.SemaphoreType.DMA,
            pltpu.SemaphoreType.DMA,
            pltpu.SemaphoreType.DMA,
            pltpu.SemaphoreType.DMA,
            pltpu.SemaphoreType.DMA,
            pltpu.SemaphoreType.DMA,
            pltpu.SemaphoreType.DMA,
            pltpu.SemaphoreType.DMA,
            pltpu.SemaphoreType.DMA,
            pltpu.SemaphoreType.DMA,
        ],
    )
    return fn(x.T, t_rm, emb_rm, jnp.squeeze(fc_table, 1), lin_w, bias_vec)


# speedup vs baseline: 1.1148x; 1.0191x over previous
import functools

import jax
import jax.numpy as jnp
import numpy as np
from jax import lax
from jax.experimental import pallas as pl
from jax.experimental.pallas import tpu as pltpu
from jax.experimental.pallas import tpu_sc as plsc

_FIELD_DIMS = np.full(27, 40000, dtype=np.int64)
_USED = _FIELD_DIMS[:-1]
_NF = len(_USED)
_D = 16
_VOCAB = int(_USED.sum())
_OFFSETS = np.concatenate([[0], np.cumsum(_USED)[:-1]]).astype(np.int32)
_B = 16384

_NW = 32
_BPW = _B // _NW
_C = 64
_NCHUNK = _BPW // _C
_CI = _C * _NF

_VB = _VOCAB // 128
_CB = 125


_NCB = (_VB + _NW - 1) // _NW
_CPS = 8
_NCHT = _NCB // _CPS
_ROWS = _CPS * 128
_ROWD = 128 * _D


def _transpose_blocks(vin, vout, idxs8, ncc):
    def ld(cc, w0, d):
        return vin[d // 8, cc, d % 8, pl.ds(w0 * 16, 16)]

    def sc(cc, w0, d, vec):
        off = cc * 2048 + w0 * 256 + 8 * (d // 8)
        plsc.store_scatter(vout.at[pl.ds(off, 256)], [idxs8[d % 8]], vec)

    groups = [(cc, w0) for cc in range(ncc) for w0 in range(8)]
    prev = [ld(*groups[0], d) for d in range(16)]
    pg = groups[0]
    for gi in range(1, len(groups)):
        cur = []
        for d in range(16):
            sc(pg[0], pg[1], d, prev[d])
            cur.append(ld(*groups[gi], d))
        prev, pg = cur, groups[gi]
    for d in range(16):
        sc(pg[0], pg[1], d, prev[d])


def _sc_transpose_body(v4_hbm, t4_hbm, out_hbm, tout_hbm,
                       vin0, vin1, vout0, vout1,
                       si0, si1, so0, so1):
    nc = 2
    wid = lax.axis_index("s") * nc + lax.axis_index("c")
    wc0 = wid * _NCB
    lanes16 = lax.iota(jnp.int32, 16) * 16
    idxs8 = [lanes16 + d for d in range(8)]
    vins = (vin0, vin1)
    vouts = (vout0, vout1)
    sis = (si0, si1)
    sos = (so0, so1)
    nch = _NCHT + 1

    def c0_of(j):
        return jnp.minimum(wc0 + j * _CPS, _VB - _CPS)

    for p in range(2):
        pltpu.async_copy(v4_hbm.at[:, pl.ds(c0_of(p), _CPS), :, :],
                         vins[p], sis[p])

    def step(k, _):
        for par in range(2):
            j = 2 * k + par
            c0 = c0_of(j)
            vin = vins[par]
            vout = vouts[par]
            so = sos[par]
            pltpu.make_async_copy(
                v4_hbm.at[:, pl.ds(c0, _CPS), :, :], vin, sis[par]).wait()

            @pl.when(k > 0)
            def _():
                pltpu.make_async_copy(
                    vout.at[pl.ds(0, _ROWD * _CPS)],
                    out_hbm.at[pl.ds(c0 * _ROWD, _ROWD * _CPS)], so).wait()

            _transpose_blocks(vin, vout, idxs8, _CPS)

            pltpu.async_copy(
                vout.at[pl.ds(0, _ROWD * _CPS)],
                out_hbm.at[pl.ds(c0 * _ROWD, _ROWD * _CPS)], so)

            @pl.when(j + 2 < nch)
            def _():
                pltpu.async_copy(
                    v4_hbm.at[:, pl.ds(c0_of(j + 2), _CPS), :, :],
                    vin, sis[par])

        return 0

    lax.fori_loop(0, nch // 2, step, 0)
    pltpu.make_async_copy(
        vout0.at[pl.ds(0, _ROWD * _CPS)],
        out_hbm.at[pl.ds(0, _ROWD * _CPS)], so0).wait()
    pltpu.make_async_copy(
        vout1.at[pl.ds(0, _ROWD * _CPS)],
        out_hbm.at[pl.ds(0, _ROWD * _CPS)], so1).wait()

    pltpu.sync_copy(t4_hbm.at[:, pl.ds(wid * 4, 4), :, :],
                    vin0.at[:, pl.ds(0, 4), :, :])
    _transpose_blocks(vin0, vout0, idxs8, 4)
    pltpu.sync_copy(vout0.at[pl.ds(0, 4 * _ROWD)],
                    tout_hbm.at[pl.ds(wid * 4 * _ROWD, 4 * _ROWD)])


def _sc_body(xT_hbm, t_hbm, emb_hbm, fc_hbm, lw_hbm, bias_hbm,
             out_hbm,
             xv0, xv1, ib0, ib1, rows0, rows1, fcv0, fcv1, tv0, tv1,
             oo0, oo1, wbuf, lw_v, bias_v,
             se0, se1, sf0, sf1, so0, so1, sx0, sx1, st0, st1):
    nc = 2
    wid = lax.axis_index("s") * nc + lax.axis_index("c")

    pltpu.sync_copy(lw_hbm, lw_v)
    pltpu.sync_copy(bias_hbm, bias_v)
    lwv = lw_v[0, :]
    bv = bias_v[0, :]
    lanes = lax.iota(jnp.int32, 16)
    lanes16 = lanes * 16
    idxs2 = [lanes16 + d for d in range(16)]
    xv = (xv0, xv1)
    ib = (ib0, ib1)
    rowsv = (rows0, rows1)
    fcv = (fcv0, fcv1)
    tv = (tv0, tv1)
    oov = (oo0, oo1)
    ses = (se0, se1)
    sfs = (sf0, sf1)
    sos = (so0, so1)
    sxs = (sx0, sx1)
    sts = (st0, st1)

    def stage_x(i, par):
        b0 = wid * _BPW + i * _C
        pltpu.async_copy(xT_hbm.at[:, pl.ds(b0, _C)], xv[par], sxs[par])

    def stage_t(i, par):
        b0 = wid * _BPW + i * _C
        pltpu.async_copy(t_hbm.at[pl.ds(b0, _C), :], tv[par], sts[par])

    def fire(i, par):
        b0 = wid * _BPW + i * _C
        pltpu.make_async_copy(
            xT_hbm.at[:, pl.ds(b0, _C)], xv[par], sxs[par]).wait()

        def build(f, _):
            off = f * 40000
            base = pl.multiple_of(f * _C, 16)
            vs = [xv[par][f, pl.ds(g * 16, 16)] + off
                  for g in range(_C // 16)]
            for g in range(_C // 16):
                ib[par][pl.ds(base + g * 16, 16)] = vs[g]
            return 0

        lax.fori_loop(0, _NF, build, 0)
        pltpu.async_copy(emb_hbm.at[ib[par]], rowsv[par], ses[par])
        pltpu.async_copy(fc_hbm.at[ib[par]], fcv[par], sfs[par])

    stage_x(0, 0)
    stage_t(0, 0)
    fire(0, 0)
    stage_x(1, 1)
    stage_t(1, 1)

    def chunk(ko, _):
        for par in range(2):
            i = 2 * ko + par
            b0 = pl.multiple_of(wid * _BPW + i * _C, _C)
            rv = rowsv[par]
            fv = fcv[par]
            tb_v = tv[par]
            pltpu.make_async_copy(emb_hbm.at[ib[par]], rv, ses[par]).wait()
            pltpu.make_async_copy(fc_hbm.at[ib[par]], fv, sfs[par]).wait()
            pltpu.make_async_copy(
                t_hbm.at[pl.ds(b0, _C), :], tb_v, sts[par]).wait()

            @pl.when(i < _NCHUNK - 1)
            def _():
                fire(i + 1, 1 - par)

            @pl.when(i < _NCHUNK - 2)
            def _():
                stage_x(i + 2, par)

            @pl.when(i >= 2)
            def _():
                pltpu.make_async_copy(
                    oov[par], out_hbm.at[pl.ds(b0, _C)], sos[par]).wait()

            for g in range(_C // 16):
                def row_body(jj, _):
                    c = g * 16 + jj * 2
                    tb0 = tb_v[c, :]
                    tb1 = tb_v[c + 1, :]
                    s0, s1 = tb0, tb1
                    q0, q1 = tb0 * tb0, tb1 * tb1
                    for f in range(_NF):
                        v0 = rv[f * _C + c, :]
                        v1 = rv[f * _C + c + 1, :]
                        s0 = s0 + v0
                        q0 = q0 + v0 * v0
                        s1 = s1 + v1
                        q1 = q1 + v1 * v1
                    w0 = 0.5 * (s0 * s0 - q0) + tb0 * lwv
                    w1 = 0.5 * (s1 * s1 - q1) + tb1 * lwv
                    j16 = pl.multiple_of(jj * 32, 16)
                    wbuf[pl.ds(j16, 16)] = w0
                    wbuf[pl.ds(j16 + 16, 16)] = w1
                    return 0

                lax.fori_loop(0, 8, row_body, 0)
                acc = plsc.load_gather(wbuf, [idxs2[0]])
                for d in range(1, 16):
                    acc = acc + plsc.load_gather(wbuf, [idxs2[d]])
                for f in range(_NF):
                    acc = acc + fv[pl.ds(f * _C + g * 16, 16)]
                oov[par][pl.ds(g * 16, 16)] = acc + bv

            @pl.when(i < _NCHUNK - 2)
            def _():
                stage_t(i + 2, par)

            pltpu.async_copy(oov[par], out_hbm.at[pl.ds(b0, _C)], sos[par])

        return 0

    lax.fori_loop(0, _NCHUNK // 2, chunk, 0)
    for par in range(2):
        pltpu.make_async_copy(
            oov[par], out_hbm.at[pl.ds(0, _C)], sos[par]).wait()


@functools.partial(jax.jit, static_argnames=())
def kernel(x, t, emb_table, fc_table, fc_bias, lin_w, lin_b):
    v4 = emb_table.reshape(_VB, 128, 2, 8).transpose(2, 0, 3, 1)
    t4 = t.reshape(128, 128, 2, 8).transpose(2, 0, 3, 1)
    mesh_t = plsc.VectorSubcoreMesh(core_axis_name="c", subcore_axis_name="s")
    emb_flat, t_flat = pl.kernel(
        _sc_transpose_body,
        out_type=(jax.ShapeDtypeStruct((_VOCAB * _D,), jnp.float32),
                  jax.ShapeDtypeStruct((_B * _D,), jnp.float32)),
        mesh=mesh_t,
        compiler_params=pltpu.CompilerParams(
            needs_layout_passes=False, use_tc_tiling_on_sc=False),
        scratch_types=[
            pltpu.VMEM((2, _CPS, 8, 128), jnp.float32),
            pltpu.VMEM((2, _CPS, 8, 128), jnp.float32),
            pltpu.VMEM((_ROWS * _D + 16,), jnp.float32),
            pltpu.VMEM((_ROWS * _D + 16,), jnp.float32),
            pltpu.SemaphoreType.DMA,
            pltpu.SemaphoreType.DMA,
            pltpu.SemaphoreType.DMA,
            pltpu.SemaphoreType.DMA,
        ],
    )(v4, t4)
    emb_rm = emb_flat.reshape(_VOCAB, _D)
    t_rm = t_flat.reshape(_B, _D)
    bias_vec = jnp.broadcast_to((fc_bias + lin_b).reshape(1, 1), (1, _D))

    mesh = plsc.VectorSubcoreMesh(core_axis_name="c", subcore_axis_name="s")
    fn = pl.kernel(
        _sc_body,
        out_type=jax.ShapeDtypeStruct((_B,), jnp.float32),
        mesh=mesh,
        compiler_params=pltpu.CompilerParams(
            needs_layout_passes=False, use_tc_tiling_on_sc=False),
        scratch_types=[
            pltpu.VMEM((_NF, _C), jnp.int32),
            pltpu.VMEM((_NF, _C), jnp.int32),
            pltpu.VMEM((_CI,), jnp.int32),
            pltpu.VMEM((_CI,), jnp.int32),
            pltpu.VMEM((_CI, _D), jnp.float32),
            pltpu.VMEM((_CI, _D), jnp.float32),
            pltpu.VMEM((_CI,), jnp.float32),
            pltpu.VMEM((_CI,), jnp.float32),
            pltpu.VMEM((_C, _D), jnp.float32),
            pltpu.VMEM((_C, _D), jnp.float32),
            pltpu.VMEM((_C,), jnp.float32),
            pltpu.VMEM((_C,), jnp.float32),
            pltpu.VMEM((256,), jnp.float32),
            pltpu.VMEM((1, _D), jnp.float32),
            pltpu.VMEM((1, _D), jnp.float32),
            pltpu.SemaphoreType.DMA,
            pltpu.SemaphoreType.DMA,
            pltpu.SemaphoreType.DMA,
            pltpu.SemaphoreType.DMA,
            pltpu.SemaphoreType.DMA,
            pltpu.SemaphoreType.DMA,
            pltpu.SemaphoreType.DMA,
            pltpu.SemaphoreType.DMA,
            pltpu.SemaphoreType.DMA,
            pltpu.SemaphoreType.DMA,
        ],
    )
    return fn(x.T, t_rm, emb_rm, jnp.squeeze(fc_table, 1), lin_w, bias_vec)
